# baseline (speedup 1.0000x reference)
import jax
import jax.numpy as jnp
from jax.experimental import pallas as pl
from jax.experimental.pallas import tpu as pltpu

_NEG_SLOPE = 0.01
_VMEM = 64 * 1024 * 1024

_BF = jnp.bfloat16
_F32 = jnp.float32


def _lrelu(x):
    return jnp.where(x > 0, x, _NEG_SLOPE * x)


def _dot(a, b):
    return jnp.dot(a, b, preferred_element_type=jnp.float32)


def _params(*sem):
    return pltpu.CompilerParams(dimension_semantics=sem,
                                vmem_limit_bytes=_VMEM)


def _gcn1_body(x_ref, a_ref, w1_ref, att_ref, h1_ref, h1b_ref, h1s_ref,
               xb_ref, ab_ref):
    @pl.when(pl.program_id(0) == 0)
    def _cast_once():
        xb_ref[...] = x_ref[...].astype(_BF)
        ab_ref[...] = a_ref[...].astype(_BF)

    t1 = _dot(xb_ref[...], w1_ref[...].astype(_BF))
    h1 = _lrelu(_dot(ab_ref[...], t1.astype(_BF)))
    h1_ref[...] = h1
    h1b_ref[...] = h1.astype(_BF)
    h1s_ref[...] = (att_ref[0] * h1).astype(_BF)


def _gcn2_body(a_ref, h1b_ref, w2_ref, att_ref, h2_ref, h2s_ref):
    ab = a_ref[...].astype(_BF)
    t2 = _dot(h1b_ref[...], w2_ref[...].astype(_BF))
    h2 = _lrelu(_dot(ab, t2.astype(_BF)))
    h2_ref[...] = h2
    h2s_ref[...] = (att_ref[0] * h2).astype(_BF)


def _y1_body(mix_ref, h1s_ref, d1t_ref, d1b_ref, y1_ref):
    y1 = _dot(mix_ref[0], d1t_ref[...]) + _dot(h1s_ref[...], d1b_ref[...])
    y1_ref[...] = y1.astype(_BF)


def _hop1_body(al_ref, ar_ref, y1t_ref, y1b_ref, d2t_ref, d2b_ref,
               h2st_ref, h2sb_ref, d1_ref, y2_ref):
    r = pl.program_id(0)
    d1 = _lrelu(_dot(al_ref[0], y1t_ref[...]) + _dot(ar_ref[0], y1b_ref[...]))
    d1_ref[0] = d1
    h2s = jnp.where(r == 0, h2st_ref[...], h2sb_ref[...])
    y2 = _dot(d1.astype(_BF), d2t_ref[...]) + _dot(h2s, d2b_ref[...])
    y2_ref[0] = y2.astype(_BF)


def _hop2_body(al_ref, ar_ref, y2_ref, d2_ref):
    d2_ref[0] = _lrelu(_dot(al_ref[0], y2_ref[0]) + _dot(ar_ref[0], y2_ref[1]))


def kernel(adj_DM, adj_D, adj_M, drg_embed, mic_embed, mix_embed,
           wd1, wd2, wm1, wm2, dm1, dm2, att):
    Nd = adj_D.shape[0]
    Nm = adj_M.shape[0]
    F = drg_embed.shape[1]
    H1 = wd1.shape[1]
    H2 = wd2.shape[1]
    N = Nd + Nm

    bf = lambda x: x.astype(_BF)

    MIX = bf(mix_embed).reshape(2, Nd, F)
    ATT = att.reshape(2, Nd, 1)
    D1T = bf(dm1[:F])
    D1B = bf(dm1[F:])
    D2T = bf(dm2[:H1])
    D2B = bf(dm2[H1:])
    AL = jnp.stack([bf(adj_DM[:Nd, :Nd]), bf(adj_DM[Nd:, :Nd])])
    AR = jnp.stack([bf(adj_DM[:Nd, Nd:]), bf(adj_DM[Nd:, Nd:])])

    TN1 = 256 if H1 % 256 == 0 else H1

    def whole(shape):
        nd = len(shape)
        return pl.BlockSpec(shape, lambda j: (0,) * nd)

    def pinned(shape, b):
        return pl.BlockSpec((1,) + shape, lambda j, _b=b: (_b, 0, 0))

    def coltile(rows, tn):
        return pl.BlockSpec((rows, tn), lambda j: (0, j))

    def gcn1(x, a, w1, b):
        return pl.pallas_call(
            _gcn1_body,
            grid=(H1 // TN1,),
            in_specs=[whole((Nd, F)), whole((Nd, Nd)), coltile(F, TN1),
                      pinned((Nd, 1), b)],
            out_specs=[coltile(Nd, TN1), coltile(Nd, TN1), coltile(Nd, TN1)],
            out_shape=[jax.ShapeDtypeStruct((Nd, H1), _F32),
                       jax.ShapeDtypeStruct((Nd, H1), _BF),
                       jax.ShapeDtypeStruct((Nd, H1), _BF)],
            scratch_shapes=[pltpu.VMEM((Nd, F), _BF),
                            pltpu.VMEM((Nd, Nd), _BF)],
            compiler_params=_params("arbitrary"),
        )(x, a, w1, ATT)

    h1_d, h1b_d, h1s_d = gcn1(drg_embed, adj_D, wd1, 0)
    h1_m, h1b_m, h1s_m = gcn1(mic_embed, adj_M, wm1, 1)

    def gcn2(a, h1b, w2, b):
        return pl.pallas_call(
            _gcn2_body,
            grid=(1,),
            in_specs=[whole((Nd, Nd)), whole((Nd, H1)), whole((H1, H2)),
                      pinned((Nd, 1), b)],
            out_specs=[whole((Nd, H2)), whole((Nd, H2))],
            out_shape=[jax.ShapeDtypeStruct((Nd, H2), _F32),
                       jax.ShapeDtypeStruct((Nd, H2), _BF)],
            compiler_params=_params("arbitrary"),
        )(a, h1b, w2, ATT)

    h2_d, h2s_d = gcn2(adj_D, h1b_d, wd2, 0)
    h2_m, h2s_m = gcn2(adj_M, h1b_m, wm2, 1)

    def y1call(h1s, b):
        return pl.pallas_call(
            _y1_body,
            grid=(H1 // TN1,),
            in_specs=[pinned((Nd, F), b), whole((Nd, H1)),
                      coltile(F, TN1), coltile(H1, TN1)],
            out_specs=coltile(Nd, TN1),
            out_shape=jax.ShapeDtypeStruct((Nd, H1), _BF),
            compiler_params=_params("arbitrary"),
        )(MIX, h1s, D1T, D1B)

    y1_t = y1call(h1s_d, 0)
    y1_b = y1call(h1s_m, 1)

    def bspec(shape):
        return pl.BlockSpec((1,) + shape, lambda r: (r, 0, 0))

    d1, y2 = pl.pallas_call(
        _hop1_body,
        grid=(2,),
        in_specs=[bspec((Nd, Nd)), bspec((Nd, Nm)), whole((Nd, H1)),
                  whole((Nd, H1)), whole((H1, H2)), whole((H2, H2)),
                  whole((Nd, H2)), whole((Nd, H2))],
        out_specs=[bspec((Nd, H1)), bspec((Nd, H2))],
        out_shape=[jax.ShapeDtypeStruct((2, Nd, H1), _F32),
                   jax.ShapeDtypeStruct((2, Nd, H2), _BF)],
        compiler_params=_params("arbitrary"),
    )(AL, AR, y1_t, y1_b, D2T, D2B, h2s_d, h2s_m)

    d2 = pl.pallas_call(
        _hop2_body,
        grid=(2,),
        in_specs=[bspec((Nd, Nd)), bspec((Nd, Nm)), whole((2, Nd, H2))],
        out_specs=bspec((Nd, H2)),
        out_shape=jax.ShapeDtypeStruct((2, Nd, H2), _F32),
        compiler_params=_params("arbitrary"),
    )(AL, AR, y2)

    return h1_d, h2_d, h1_m, h2_m, d1.reshape(N, H1), d2.reshape(N, H2)


# speedup vs baseline: 1.1074x; 1.1074x over previous
import jax
import jax.numpy as jnp
from jax.experimental import pallas as pl
from jax.experimental.pallas import tpu as pltpu

_NEG_SLOPE = 0.01
_VMEM = 64 * 1024 * 1024

_BF = jnp.bfloat16
_F32 = jnp.float32


def _lrelu(x):
    return jnp.where(x > 0, x, _NEG_SLOPE * x)


def _dot(a, b):
    return jnp.dot(a, b, preferred_element_type=jnp.float32)


def _branch_body(x_ref, a_ref, w1_ref, w2_ref, mix_ref, d1t_ref, d1b_ref,
                 att_ref, h1d_ref, h1m_ref, h2d_ref, h2m_ref, y1_ref,
                 h2s_ref):
    b = pl.program_id(0)
    a = a_ref[0]
    att = att_ref[0]

    t1 = _dot(x_ref[0], w1_ref[0]).astype(_BF)
    h1 = _lrelu(_dot(a, t1))
    h1s = (att * h1).astype(_BF)

    t2 = _dot(h1.astype(_BF), w2_ref[0]).astype(_BF)
    h2 = _lrelu(_dot(a, t2))
    h2s_ref[0] = (att * h2).astype(_BF)

    @pl.when(b == 0)
    def _drug_out():
        h1d_ref[...] = h1
        h2d_ref[...] = h2

    @pl.when(b == 1)
    def _microbe_out():
        h1m_ref[...] = h1
        h2m_ref[...] = h2

    y1 = _dot(mix_ref[0], d1t_ref[...]) + _dot(h1s, d1b_ref[...])
    y1_ref[0] = y1.astype(_BF)


def _hetero_body(al_ref, ar_ref, y1_ref, d2t_ref, d2b_ref, h2s_ref,
                 d1_ref, d2_ref, y2s_ref):
    j = pl.program_id(0)

    @pl.when(j < 2)
    def _hop1():
        d1 = _lrelu(_dot(al_ref[0], y1_ref[0]) + _dot(ar_ref[0], y1_ref[1]))
        d1_ref[0] = d1
        y2 = (_dot(d1.astype(_BF), d2t_ref[...]) +
              _dot(h2s_ref[0], d2b_ref[...])).astype(_BF)

        @pl.when(j == 0)
        def _store_top():
            y2s_ref[0] = y2

        @pl.when(j == 1)
        def _store_bot():
            y2s_ref[1] = y2

    @pl.when(j >= 2)
    def _hop2():
        d2_ref[0] = _lrelu(_dot(al_ref[0], y2s_ref[0]) +
                           _dot(ar_ref[0], y2s_ref[1]))


def kernel(adj_DM, adj_D, adj_M, drg_embed, mic_embed, mix_embed,
           wd1, wd2, wm1, wm2, dm1, dm2, att):
    Nd = adj_D.shape[0]
    Nm = adj_M.shape[0]
    F = drg_embed.shape[1]
    H1 = wd1.shape[1]
    H2 = wd2.shape[1]
    N = Nd + Nm

    bf = lambda x: x.astype(_BF)

    X = jnp.stack([bf(drg_embed), bf(mic_embed)])
    A = jnp.stack([bf(adj_D), bf(adj_M)])
    W1 = jnp.stack([bf(wd1), bf(wm1)])
    W2 = jnp.stack([bf(wd2), bf(wm2)])
    MIX = bf(mix_embed).reshape(2, Nd, F)
    ATT = att.reshape(2, Nd, 1)
    D1T = bf(dm1[:F])
    D1B = bf(dm1[F:])
    D2T = bf(dm2[:H1])
    D2B = bf(dm2[H1:])
    AL = jnp.stack([bf(adj_DM[:Nd, :Nd]), bf(adj_DM[Nd:, :Nd])])
    AR = jnp.stack([bf(adj_DM[:Nd, Nd:]), bf(adj_DM[Nd:, Nd:])])

    def bspec(shape):
        return pl.BlockSpec((1,) + shape, lambda b: (b, 0, 0))

    def whole(shape):
        nd = len(shape)
        return pl.BlockSpec(shape, lambda b: (0,) * nd)

    params = pltpu.CompilerParams(dimension_semantics=("arbitrary",),
                                  vmem_limit_bytes=_VMEM)

    h1d, h1m, h2d, h2m, y1, h2s = pl.pallas_call(
        _branch_body,
        grid=(2,),
        in_specs=[bspec((Nd, F)), bspec((Nd, Nd)), bspec((F, H1)),
                  bspec((H1, H2)), bspec((Nd, F)), whole((F, H1)),
                  whole((H1, H1)), bspec((Nd, 1))],
        out_specs=[whole((Nd, H1)), whole((Nm, H1)), whole((Nd, H2)),
                   whole((Nm, H2)), bspec((Nd, H1)), bspec((Nd, H2))],
        out_shape=[jax.ShapeDtypeStruct((Nd, H1), _F32),
                   jax.ShapeDtypeStruct((Nm, H1), _F32),
                   jax.ShapeDtypeStruct((Nd, H2), _F32),
                   jax.ShapeDtypeStruct((Nm, H2), _F32),
                   jax.ShapeDtypeStruct((2, Nd, H1), _BF),
                   jax.ShapeDtypeStruct((2, Nd, H2), _BF)],
        compiler_params=params,
    )(X, A, W1, W2, MIX, D1T, D1B, ATT)

    adj_idx = lambda j: (jnp.minimum(j, 3 - j), 0, 0)
    d1, d2 = pl.pallas_call(
        _hetero_body,
        grid=(4,),
        in_specs=[pl.BlockSpec((1, Nd, Nd), adj_idx),
                  pl.BlockSpec((1, Nd, Nm), adj_idx),
                  whole((2, Nd, H1)), whole((H1, H2)), whole((H2, H2)),
                  pl.BlockSpec((1, Nd, H2),
                               lambda j: (jnp.minimum(j, 1), 0, 0))],
        out_specs=[pl.BlockSpec((1, Nd, H1),
                                lambda j: (jnp.minimum(j, 1), 0, 0)),
                   pl.BlockSpec((1, Nd, H2),
                                lambda j: (jnp.minimum(3 - j, 1), 0, 0))],
        out_shape=[jax.ShapeDtypeStruct((2, Nd, H1), _F32),
                   jax.ShapeDtypeStruct((2, Nd, H2), _F32)],
        scratch_shapes=[pltpu.VMEM((2, Nd, H2), _BF)],
        compiler_params=params,
    )(AL, AR, y1, D2T, D2B, h2s)

    return h1d, h2d, h1m, h2m, d1.reshape(N, H1), d2.reshape(N, H2)


# speedup vs baseline: 1.1209x; 1.0122x over previous
import jax
import jax.numpy as jnp
from jax.experimental import pallas as pl
from jax.experimental.pallas import tpu as pltpu

_NEG_SLOPE = 0.01
_VMEM = 64 * 1024 * 1024

_BF = jnp.bfloat16
_F32 = jnp.float32


def _lrelu(x):
    return jnp.where(x > 0, x, _NEG_SLOPE * x)


def _dot(a, b):
    return jnp.dot(a, b, preferred_element_type=jnp.float32)


def _main_body(x_ref, a_ref, w1_ref, w2_ref, mix_ref, d1t_ref, d1b_ref,
               att_ref, al_ref, ar_ref, d2t_ref, d2b_ref,
               h1d_ref, h1m_ref, h2d_ref, h2m_ref, d1_ref, y2_ref,
               y1s_ref, h2ss_ref):
    j = pl.program_id(0)

    @pl.when(j < 2)
    def _branch():
        a = a_ref[0]
        att = att_ref[0]

        t1 = _dot(x_ref[0], w1_ref[0]).astype(_BF)
        h1 = _lrelu(_dot(a, t1))
        h1s = (att * h1).astype(_BF)

        t2 = _dot(h1.astype(_BF), w2_ref[0]).astype(_BF)
        h2 = _lrelu(_dot(a, t2))
        h2s = (att * h2).astype(_BF)

        y1 = (_dot(mix_ref[0], d1t_ref[...]) +
              _dot(h1s, d1b_ref[...])).astype(_BF)

        @pl.when(j == 0)
        def _drug():
            h1d_ref[...] = h1
            h2d_ref[...] = h2
            y1s_ref[0] = y1
            h2ss_ref[0] = h2s

        @pl.when(j == 1)
        def _microbe():
            h1m_ref[...] = h1
            h2m_ref[...] = h2
            y1s_ref[1] = y1
            h2ss_ref[1] = h2s

    @pl.when(j >= 2)
    def _hop1():
        d1 = _lrelu(_dot(al_ref[0], y1s_ref[0]) + _dot(ar_ref[0], y1s_ref[1]))
        d1_ref[0] = d1

        @pl.when(j == 2)
        def _top():
            y2_ref[0] = (_dot(d1.astype(_BF), d2t_ref[...]) +
                         _dot(h2ss_ref[0], d2b_ref[...])).astype(_BF)

        @pl.when(j == 3)
        def _bot():
            y2_ref[0] = (_dot(d1.astype(_BF), d2t_ref[...]) +
                         _dot(h2ss_ref[1], d2b_ref[...])).astype(_BF)


def _hop2_body(al_ref, ar_ref, y2_ref, d2_ref):
    d2_ref[0] = _lrelu(_dot(al_ref[0], y2_ref[0]) + _dot(ar_ref[0], y2_ref[1]))


def kernel(adj_DM, adj_D, adj_M, drg_embed, mic_embed, mix_embed,
           wd1, wd2, wm1, wm2, dm1, dm2, att):
    Nd = adj_D.shape[0]
    Nm = adj_M.shape[0]
    F = drg_embed.shape[1]
    H1 = wd1.shape[1]
    H2 = wd2.shape[1]
    N = Nd + Nm

    bf = lambda x: x.astype(_BF)

    X = jnp.stack([bf(drg_embed), bf(mic_embed)])
    A = jnp.stack([bf(adj_D), bf(adj_M)])
    W1 = jnp.stack([bf(wd1), bf(wm1)])
    W2 = jnp.stack([bf(wd2), bf(wm2)])
    MIX = bf(mix_embed).reshape(2, Nd, F)
    ATT = att.reshape(2, Nd, 1)
    D1T = bf(dm1[:F])
    D1B = bf(dm1[F:])
    D2T = bf(dm2[:H1])
    D2B = bf(dm2[H1:])
    AL = jnp.stack([bf(adj_DM[:Nd, :Nd]), bf(adj_DM[Nd:, :Nd])])
    AR = jnp.stack([bf(adj_DM[:Nd, Nd:]), bf(adj_DM[Nd:, Nd:])])

    def branch_blk(shape):
        return pl.BlockSpec((1,) + shape, lambda j: (jnp.minimum(j, 1), 0, 0))

    def hop_blk(shape):
        return pl.BlockSpec((1,) + shape,
                            lambda j: (jnp.clip(j - 2, 0, 1), 0, 0))

    def whole(shape):
        nd = len(shape)
        return pl.BlockSpec(shape, lambda j: (0,) * nd)

    params = pltpu.CompilerParams(dimension_semantics=("arbitrary",),
                                  vmem_limit_bytes=_VMEM)

    h1d, h1m, h2d, h2m, d1, y2 = pl.pallas_call(
        _main_body,
        grid=(4,),
        in_specs=[branch_blk((Nd, F)), branch_blk((Nd, Nd)),
                  branch_blk((F, H1)), branch_blk((H1, H2)),
                  branch_blk((Nd, F)), whole((F, H1)), whole((H1, H1)),
                  branch_blk((Nd, 1)), hop_blk((Nd, Nd)), hop_blk((Nd, Nm)),
                  whole((H1, H2)), whole((H2, H2))],
        out_specs=[whole((Nd, H1)), whole((Nm, H1)), whole((Nd, H2)),
                   whole((Nm, H2)), hop_blk((Nd, H1)), hop_blk((Nd, H2))],
        out_shape=[jax.ShapeDtypeStruct((Nd, H1), _F32),
                   jax.ShapeDtypeStruct((Nm, H1), _F32),
                   jax.ShapeDtypeStruct((Nd, H2), _F32),
                   jax.ShapeDtypeStruct((Nm, H2), _F32),
                   jax.ShapeDtypeStruct((2, Nd, H1), _F32),
                   jax.ShapeDtypeStruct((2, Nd, H2), _BF)],
        scratch_shapes=[pltpu.VMEM((2, Nd, H1), _BF),
                        pltpu.VMEM((2, Nd, H2), _BF)],
        compiler_params=params,
    )(X, A, W1, W2, MIX, D1T, D1B, ATT, AL, AR, D2T, D2B)

    d2 = pl.pallas_call(
        _hop2_body,
        grid=(2,),
        in_specs=[pl.BlockSpec((1, Nd, Nd), lambda r: (r, 0, 0)),
                  pl.BlockSpec((1, Nd, Nm), lambda r: (r, 0, 0)),
                  whole((2, Nd, H2))],
        out_specs=pl.BlockSpec((1, Nd, H2), lambda r: (r, 0, 0)),
        out_shape=jax.ShapeDtypeStruct((2, Nd, H2), _F32),
        compiler_params=params,
    )(AL, AR, y2)

    return h1d, h2d, h1m, h2m, d1.reshape(N, H1), d2.reshape(N, H2)
